# baseline (speedup 1.0000x reference)
import functools

import numpy as np
import jax
import jax.numpy as jnp
from jax.experimental import pallas as pl
from jax.experimental.pallas import tpu as pltpu

_N = 4096
_NT = 32
_BINS = 28
_B = 1024
_NBLK = _N // _B
_NSTEP = _NBLK * (_NBLK + 1) // 2
_INV_BIN = 1.0 / 0.7
_DCUT = 19.6
_SEP_EXCL = 2.0

_BMAP = np.array(
    [[bi, bj] for bi in range(_NBLK) for bj in range(bi, _NBLK)],
    dtype=np.int32).T


def _score_kernel(bmap, xi, xjt, ri, rjt, tit, tj, pot, out, hist, ustack):
    g = pl.program_id(0)
    bi = bmap[0, g]
    bj = bmap[1, g]

    @pl.when(g == 0)
    def _init():
        hist[...] = jnp.zeros_like(hist)

    d2 = jnp.square(xi[:, 0:1] - xjt[0:1, :])
    d2 = d2 + jnp.square(xi[:, 1:2] - xjt[1:2, :])
    d2 = d2 + jnp.square(xi[:, 2:3] - xjt[2:3, :])
    dist = jnp.sqrt(d2) + 1e-8

    sep = jnp.abs(ri[...] - rjt[...])
    mask = (sep > _SEP_EXCL) & (dist < _DCUT)

    ds = dist * _INV_BIN
    d0f = jnp.floor(jnp.minimum(ds, 27.0))
    alpha = ds - d0f
    scale = jnp.where(bi == bj, 0.5, 1.0)
    wa = jnp.where(mask, scale, 0.0)
    w1 = wa * alpha
    d0m = jnp.where(mask, d0f, -1.0)

    zero = jnp.zeros_like(wa)
    d_prev = None
    for b in range(_BINS):
        m_b = d0m == float(b)
        e_op = jnp.where(m_b, scale, 0.0)
        d_op = jnp.where(m_b, w1, zero)
        e_b = jax.lax.dot_general(
            tit[...], e_op, (((1,), (0,)), ((), ())),
            preferred_element_type=jnp.float32)
        d_b = jax.lax.dot_general(
            tit[...], d_op, (((1,), (0,)), ((), ())),
            preferred_element_type=jnp.float32)
        if b == 0:
            u = e_b - d_b
        elif b < _BINS - 1:
            u = e_b - d_b + d_prev
        else:
            u = e_b + d_prev
        d_prev = d_b
        ustack[b * _NT:(b + 1) * _NT, :] = u
    us = ustack[...]
    uh = us.astype(jnp.bfloat16)
    ul = (us - uh.astype(jnp.float32)).astype(jnp.bfloat16)
    m32 = jax.lax.dot_general(
        uh, tj[...], (((1,), (0,)), ((), ())),
        preferred_element_type=jnp.float32)
    m32 = m32 + jax.lax.dot_general(
        ul, tj[...], (((1,), (0,)), ((), ())),
        preferred_element_type=jnp.float32)
    hist[...] += m32

    @pl.when(g == _NSTEP - 1)
    def _fin():
        out[...] = jnp.sum(hist[...] * pot[...], keepdims=True)


@jax.jit
def _run(x, xt, rf, rft, tit, tj, pot_r):
    out = pl.pallas_call(
        _score_kernel,
        grid_spec=pltpu.PrefetchScalarGridSpec(
            num_scalar_prefetch=1,
            grid=(_NSTEP,),
            in_specs=[
                pl.BlockSpec((_B, 8), lambda g, m: (m[0, g], 0)),
                pl.BlockSpec((8, _B), lambda g, m: (0, m[1, g])),
                pl.BlockSpec((_B, 1), lambda g, m: (m[0, g], 0)),
                pl.BlockSpec((1, _B), lambda g, m: (0, m[1, g])),
                pl.BlockSpec((_NT, _B), lambda g, m: (0, m[0, g])),
                pl.BlockSpec((_B, _NT), lambda g, m: (m[1, g], 0)),
                pl.BlockSpec((_BINS * _NT, _NT), lambda g, m: (0, 0)),
            ],
            out_specs=pl.BlockSpec((1, 1), lambda g, m: (0, 0)),
            scratch_shapes=[
                pltpu.VMEM((_BINS * _NT, _NT), jnp.float32),
                pltpu.VMEM((_BINS * _NT, _B), jnp.float32),
            ],
        ),
        out_shape=jax.ShapeDtypeStruct((1, 1), jnp.float32),
    )(jnp.asarray(_BMAP), x, xt, rf, rft, tit, tj, pot_r)
    return out[0, 0]


def kernel(coords, type_indices, res_ids, pot_tensor):
    x = jnp.pad(coords.astype(jnp.float32), ((0, 0), (0, 5)))
    xt = x.T
    rf = res_ids.astype(jnp.float32).reshape(_N, 1)
    rft = rf.T
    onehot = (type_indices.reshape(_N, 1) ==
              jnp.arange(_NT, dtype=type_indices.dtype).reshape(1, _NT)
              ).astype(jnp.float32)
    tit = onehot.T
    pot_r = jnp.transpose(pot_tensor.astype(jnp.float32), (2, 0, 1)).reshape(
        _BINS * _NT, _NT)
    return _run(x, xt, rf, rft, tit, onehot, pot_r)


# speedup vs baseline: 1.0558x; 1.0558x over previous
import functools

import numpy as np
import jax
import jax.numpy as jnp
from jax.experimental import pallas as pl
from jax.experimental.pallas import tpu as pltpu

_N = 4096
_NT = 32
_BINS = 28
_B = 512
_NBLK = _N // _B
_NSTEP = _NBLK * (_NBLK + 1) // 2
_INV_BIN = 1.0 / 0.7
_DCUT = 19.6
_SEP_EXCL = 2.0

_BMAP = np.array(
    [[bi, bj] for bi in range(_NBLK) for bj in range(bi, _NBLK)],
    dtype=np.int32).T


def _score_kernel(bmap, xi, xjt, ri, rjt, tit, tj, pot, out, hist, ustack):
    g = pl.program_id(0)
    bi = bmap[0, g]
    bj = bmap[1, g]

    @pl.when(g == 0)
    def _init():
        hist[...] = jnp.zeros_like(hist)

    d2 = jnp.square(xi[:, 0:1] - xjt[0:1, :])
    d2 = d2 + jnp.square(xi[:, 1:2] - xjt[1:2, :])
    d2 = d2 + jnp.square(xi[:, 2:3] - xjt[2:3, :])
    dist = jnp.sqrt(d2) + 1e-8

    sep = jnp.abs(ri[...] - rjt[...])
    mask = (sep > _SEP_EXCL) & (dist < _DCUT)

    ds = dist * _INV_BIN
    d0f = jnp.floor(jnp.minimum(ds, 27.0))
    alpha = ds - d0f
    scale = jnp.where(bi == bj, 0.5, 1.0)
    wa = jnp.where(mask, scale, 0.0)
    w1 = wa * alpha
    d0m = jnp.where(mask, d0f, -1.0).astype(jnp.bfloat16)
    w1b = w1.astype(jnp.bfloat16)
    scale_b = scale.astype(jnp.bfloat16)

    zero = jnp.zeros_like(w1b)
    d_prev = None
    for b in range(_BINS):
        m_b = d0m == jnp.bfloat16(b)
        e_op = jnp.where(m_b, scale_b, jnp.bfloat16(0))
        d_op = jnp.where(m_b, w1b, zero)
        e_b = jax.lax.dot_general(
            tit[...], e_op, (((1,), (0,)), ((), ())),
            preferred_element_type=jnp.float32)
        d_b = jax.lax.dot_general(
            tit[...], d_op, (((1,), (0,)), ((), ())),
            preferred_element_type=jnp.float32)
        if b == 0:
            u = e_b - d_b
        elif b < _BINS - 1:
            u = e_b - d_b + d_prev
        else:
            u = e_b + d_prev
        d_prev = d_b
        ustack[b * _NT:(b + 1) * _NT, :] = u
    us = ustack[...]
    uh = us.astype(jnp.bfloat16)
    ul = (us - uh.astype(jnp.float32)).astype(jnp.bfloat16)
    m32 = jax.lax.dot_general(
        uh, tj[...], (((1,), (0,)), ((), ())),
        preferred_element_type=jnp.float32)
    m32 = m32 + jax.lax.dot_general(
        ul, tj[...], (((1,), (0,)), ((), ())),
        preferred_element_type=jnp.float32)
    hist[...] += m32

    @pl.when(g == _NSTEP - 1)
    def _fin():
        out[...] = jnp.sum(hist[...] * pot[...], keepdims=True)


@jax.jit
def _run(x, xt, rf, rft, tit, tj, pot_r):
    out = pl.pallas_call(
        _score_kernel,
        grid_spec=pltpu.PrefetchScalarGridSpec(
            num_scalar_prefetch=1,
            grid=(_NSTEP,),
            in_specs=[
                pl.BlockSpec((_B, 8), lambda g, m: (m[0, g], 0)),
                pl.BlockSpec((8, _B), lambda g, m: (0, m[1, g])),
                pl.BlockSpec((_B, 1), lambda g, m: (m[0, g], 0)),
                pl.BlockSpec((1, _B), lambda g, m: (0, m[1, g])),
                pl.BlockSpec((_NT, _B), lambda g, m: (0, m[0, g])),
                pl.BlockSpec((_B, _NT), lambda g, m: (m[1, g], 0)),
                pl.BlockSpec((_BINS * _NT, _NT), lambda g, m: (0, 0)),
            ],
            out_specs=pl.BlockSpec((1, 1), lambda g, m: (0, 0)),
            scratch_shapes=[
                pltpu.VMEM((_BINS * _NT, _NT), jnp.float32),
                pltpu.VMEM((_BINS * _NT, _B), jnp.float32),
            ],
        ),
        out_shape=jax.ShapeDtypeStruct((1, 1), jnp.float32),
    )(jnp.asarray(_BMAP), x, xt, rf, rft, tit, tj, pot_r)
    return out[0, 0]


def kernel(coords, type_indices, res_ids, pot_tensor):
    x = jnp.pad(coords.astype(jnp.float32), ((0, 0), (0, 5)))
    xt = x.T
    rf = res_ids.astype(jnp.float32).reshape(_N, 1)
    rft = rf.T
    onehot = (type_indices.reshape(_N, 1) ==
              jnp.arange(_NT, dtype=type_indices.dtype).reshape(1, _NT)
              ).astype(jnp.bfloat16)
    tit = onehot.T
    pot_r = jnp.transpose(pot_tensor.astype(jnp.float32), (2, 0, 1)).reshape(
        _BINS * _NT, _NT)
    return _run(x, xt, rf, rft, tit, onehot, pot_r)
